# baseline (speedup 1.0000x reference)
import functools

import jax
import jax.numpy as jnp
from jax import lax
from jax.experimental import pallas as pl
from jax.experimental.pallas import tpu as pltpu
from jax.experimental.pallas import tpu_sc as plsc

_LANES = 16


def _sc_scatter_partials(senders, receivers, edge_weight, n, nblk, blk):
    e = senders.shape[0]
    info = plsc.get_sparse_core_info()
    nc, ns = info.num_cores, info.num_subcores
    nw = nc * ns
    assert e % (nw * _LANES) == 0
    ept = e // nw
    groups = ept // _LANES
    mesh = plsc.VectorSubcoreMesh(core_axis_name="c", subcore_axis_name="s")

    @functools.partial(
        pl.kernel,
        mesh=mesh,
        compiler_params=pltpu.CompilerParams(needs_layout_passes=False),
        out_type=(
            jax.ShapeDtypeStruct((nw, n), jnp.float32),
            jax.ShapeDtypeStruct((nw, n), jnp.float32),
        ),
        scratch_types=[
            pltpu.VMEM((ept,), jnp.int32),
            pltpu.VMEM((ept,), jnp.int32),
            pltpu.VMEM((ept,), jnp.float32),
            pltpu.VMEM((n,), jnp.float32),
            pltpu.VMEM((n,), jnp.float32),
            pltpu.SemaphoreType.DMA,
            pltpu.SemaphoreType.DMA,
            pltpu.SemaphoreType.DMA,
        ],
    )
    def scatter_kernel(send_hbm, recv_hbm, w_hbm, degp_hbm, sp_hbm,
                       send_v, recv_v, w_v, dacc, sacc, sem0, sem1, sem2):
        wid = lax.axis_index("s") * nc + lax.axis_index("c")
        base = wid * ept
        cp0 = pltpu.async_copy(send_hbm.at[pl.ds(base, ept)], send_v, sem0)
        cp1 = pltpu.async_copy(recv_hbm.at[pl.ds(base, ept)], recv_v, sem1)
        cp2 = pltpu.async_copy(w_hbm.at[pl.ds(base, ept)], w_v, sem2)

        zeros = jnp.zeros((_LANES,), jnp.float32)

        @plsc.parallel_loop(0, n, _LANES, unroll=8)
        def _zero(j):
            dacc[pl.ds(j, _LANES)] = zeros
            sacc[pl.ds(j, _LANES)] = zeros

        cp0.wait()
        cp1.wait()
        cp2.wait()

        @plsc.parallel_loop(0, ept, _LANES, unroll=16)
        def _scatter(j):
            sl = pl.ds(j, _LANES)
            w = w_v[sl]
            plsc.addupdate_scatter(dacc, [send_v[sl]], w)
            plsc.addupdate_scatter(sacc, [recv_v[sl]], w)

        ocp0 = pltpu.async_copy(dacc, degp_hbm.at[wid], sem0)
        ocp1 = pltpu.async_copy(sacc, sp_hbm.at[wid], sem1)
        ocp0.wait()
        ocp1.wait()

    return scatter_kernel(senders, receivers, edge_weight)


_BLK_PAD = 2048


def _tc_body(k, blk, nblk, degp_ref, sp_ref, x_ref, ws_ref, bs_ref, bias_ref,
             out_ref, r_scr, wa_scr, wb_scr):
    i = pl.program_id(0)

    @pl.when(i == 0)
    def _():
        deg = jnp.sum(degp_ref[...], axis=0)
        s = jnp.sum(sp_ref[...], axis=0)
        r_all = (deg - s) / jnp.max(deg) - 1.0
        for b in range(nblk):
            r_scr[0, b * _BLK_PAD:b * _BLK_PAD + blk] = (
                r_all[b * blk:(b + 1) * blk])
        wa = ws_ref[0]
        wb = jnp.zeros_like(wa)
        for kk in range(1, k):
            wa = wa + ws_ref[kk]
            wb = wb + float(kk) * ws_ref[kk]
        wa_scr[...] = wa
        wb_scr[...] = wb

    rm1 = r_scr[0, pl.ds(i * _BLK_PAD, blk)]
    xb = x_ref[...]
    bsum = jnp.sum(bs_ref[...], axis=0, keepdims=True) + bias_ref[...]
    acc = (jnp.dot(xb, wa_scr[...], preferred_element_type=jnp.float32)
           + rm1[:, None]
           * jnp.dot(xb, wb_scr[...], preferred_element_type=jnp.float32)
           + bsum)
    out_ref[...] = acc


def kernel(x, senders, receivers, edge_weight, Ws, bs, bias):
    n, d = x.shape
    k, _, out_dim = Ws.shape
    blk = 1000
    assert n % blk == 0
    nblk = n // blk

    degp, sp = _sc_scatter_partials(senders, receivers, edge_weight, n,
                                    nblk, blk)
    nw = degp.shape[0]

    return pl.pallas_call(
        functools.partial(_tc_body, k, blk, nblk),
        grid=(nblk,),
        in_specs=[
            pl.BlockSpec((nw, n), lambda i: (0, 0)),
            pl.BlockSpec((nw, n), lambda i: (0, 0)),
            pl.BlockSpec((blk, d), lambda i: (i, 0)),
            pl.BlockSpec((k, d, out_dim), lambda i: (0, 0, 0)),
            pl.BlockSpec((k, out_dim), lambda i: (0, 0)),
            pl.BlockSpec((1, out_dim), lambda i: (0, 0)),
        ],
        out_specs=pl.BlockSpec((blk, out_dim), lambda i: (i, 0)),
        out_shape=jax.ShapeDtypeStruct((n, out_dim), jnp.float32),
        scratch_shapes=[
            pltpu.VMEM((1, nblk * _BLK_PAD), jnp.float32),
            pltpu.VMEM((d, out_dim), jnp.float32),
            pltpu.VMEM((d, out_dim), jnp.float32),
        ],
    )(degp, sp, x, Ws, bs, bias.reshape(1, out_dim))


# speedup vs baseline: 1.0778x; 1.0778x over previous
import functools

import jax
import jax.numpy as jnp
from jax import lax
from jax.experimental import pallas as pl
from jax.experimental.pallas import tpu as pltpu
from jax.experimental.pallas import tpu_sc as plsc

_LANES = 16


def _sc_scatter_partials(senders, receivers, edge_weight, n, nblk, blk):
    e = senders.shape[0]
    info = plsc.get_sparse_core_info()
    nc, ns = info.num_cores, info.num_subcores
    nw = nc * ns
    assert e % (nw * _LANES) == 0
    ept = e // nw
    groups = ept // _LANES
    mesh = plsc.VectorSubcoreMesh(core_axis_name="c", subcore_axis_name="s")

    @functools.partial(
        pl.kernel,
        mesh=mesh,
        compiler_params=pltpu.CompilerParams(needs_layout_passes=False),
        out_type=(
            jax.ShapeDtypeStruct((nw, n), jnp.float32),
            jax.ShapeDtypeStruct((nw, n), jnp.float32),
        ),
        scratch_types=[
            pltpu.VMEM((ept,), jnp.int32),
            pltpu.VMEM((ept,), jnp.int32),
            pltpu.VMEM((ept,), jnp.float32),
            pltpu.VMEM((n,), jnp.float32),
            pltpu.VMEM((n,), jnp.float32),
            pltpu.SemaphoreType.DMA,
            pltpu.SemaphoreType.DMA,
            pltpu.SemaphoreType.DMA,
        ],
    )
    def scatter_kernel(send_hbm, recv_hbm, w_hbm, degp_hbm, sp_hbm,
                       send_v, recv_v, w_v, dacc, sacc, sem0, sem1, sem2):
        wid = lax.axis_index("s") * nc + lax.axis_index("c")
        base = wid * ept
        cp0 = pltpu.async_copy(send_hbm.at[pl.ds(base, ept)], send_v, sem0)
        cp1 = pltpu.async_copy(recv_hbm.at[pl.ds(base, ept)], recv_v, sem1)
        cp2 = pltpu.async_copy(w_hbm.at[pl.ds(base, ept)], w_v, sem2)

        zeros = jnp.zeros((_LANES,), jnp.float32)

        @plsc.parallel_loop(0, n, _LANES, unroll=8)
        def _zero(j):
            dacc[pl.ds(j, _LANES)] = zeros
            sacc[pl.ds(j, _LANES)] = zeros

        cp0.wait()
        cp1.wait()
        cp2.wait()

        @plsc.parallel_loop(0, ept, _LANES, unroll=16)
        def _scatter(j):
            sl = pl.ds(j, _LANES)
            w = w_v[sl]
            plsc.addupdate_scatter(dacc, [send_v[sl]], w)
            plsc.addupdate_scatter(sacc, [recv_v[sl]], w)

        ocp0 = pltpu.async_copy(dacc, degp_hbm.at[wid], sem0)
        ocp1 = pltpu.async_copy(sacc, sp_hbm.at[wid], sem1)
        ocp0.wait()
        ocp1.wait()

    return scatter_kernel(senders, receivers, edge_weight)


def _blk_pad(blk):
    return ((blk + 127) // 128) * 128


def _tc_body(k, blk, nblk, degp_ref, sp_ref, x_ref, ws_ref, bs_ref, bias_ref,
             out_ref, r_scr, wa_scr, wb_scr):
    pad = _blk_pad(blk)
    i = pl.program_id(0)

    @pl.when(i == 0)
    def _():
        deg = jnp.sum(degp_ref[...], axis=0)
        s = jnp.sum(sp_ref[...], axis=0)
        r_all = (deg - s) / jnp.max(deg) - 1.0
        for b in range(nblk):
            r_scr[0, b * pad:b * pad + blk] = (
                r_all[b * blk:(b + 1) * blk])
        wa = ws_ref[0]
        wb = jnp.zeros_like(wa)
        for kk in range(1, k):
            wa = wa + ws_ref[kk]
            wb = wb + float(kk) * ws_ref[kk]
        wa_scr[...] = wa
        wb_scr[...] = wb

    rm1 = r_scr[0, pl.ds(i * pad, blk)]
    xb = x_ref[...]
    bsum = jnp.sum(bs_ref[...], axis=0, keepdims=True) + bias_ref[...]
    acc = (jnp.dot(xb, wa_scr[...], preferred_element_type=jnp.float32)
           + rm1[:, None]
           * jnp.dot(xb, wb_scr[...], preferred_element_type=jnp.float32)
           + bsum)
    out_ref[...] = acc


def kernel(x, senders, receivers, edge_weight, Ws, bs, bias):
    n, d = x.shape
    k, _, out_dim = Ws.shape
    blk = 2000
    assert n % blk == 0
    nblk = n // blk

    degp, sp = _sc_scatter_partials(senders, receivers, edge_weight, n,
                                    nblk, blk)
    nw = degp.shape[0]

    return pl.pallas_call(
        functools.partial(_tc_body, k, blk, nblk),
        grid=(nblk,),
        in_specs=[
            pl.BlockSpec((nw, n), lambda i: (0, 0)),
            pl.BlockSpec((nw, n), lambda i: (0, 0)),
            pl.BlockSpec((blk, d), lambda i: (i, 0)),
            pl.BlockSpec((k, d, out_dim), lambda i: (0, 0, 0)),
            pl.BlockSpec((k, out_dim), lambda i: (0, 0)),
            pl.BlockSpec((1, out_dim), lambda i: (0, 0)),
        ],
        out_specs=pl.BlockSpec((blk, out_dim), lambda i: (i, 0)),
        out_shape=jax.ShapeDtypeStruct((n, out_dim), jnp.float32),
        scratch_shapes=[
            pltpu.VMEM((1, nblk * _blk_pad(blk)), jnp.float32),
            pltpu.VMEM((d, out_dim), jnp.float32),
            pltpu.VMEM((d, out_dim), jnp.float32),
        ],
    )(degp, sp, x, Ws, bs, bias.reshape(1, out_dim))
